# baseline (speedup 1.0000x reference)
import dataclasses
import functools

import jax
import jax.numpy as jnp
import numpy as np
from jax.experimental import pallas as pl
from jax.experimental.pallas import tpu as pltpu
from jax.experimental.pallas import tpu_sc as plsc

N = 10000
E = 320000
D_OUT = 128
H = 8

_NP = 10240
_CQ = 128

_vector_mesh = plsc.VectorSubcoreMesh(
    core_axis_name="core", subcore_axis_name="subcore")

_sc_params = pltpu.CompilerParams()
if "needs_layout_passes" in pltpu.CompilerParams.__dataclass_fields__:
  _sc_params = dataclasses.replace(_sc_params, needs_layout_passes=False)


def _pack_perms(width):
  q = np.arange(width // 2)
  lo = 32 * (q // 16) + q % 16
  return np.asarray(lo, np.int32), np.asarray(lo + 16, np.int32)


def _linear(x, w, b, out_dtype=jnp.float32, block_rows=512):
  R, K = x.shape
  Dout = w.shape[1]

  def body(x_ref, w_ref, b_ref, o_ref):
    o_ref[...] = (jnp.dot(x_ref[...], w_ref[...],
                          preferred_element_type=jnp.float32)
                  + b_ref[...]).astype(out_dtype)

  br = min(block_rows, R)
  return pl.pallas_call(
      body,
      grid=(pl.cdiv(R, br),),
      in_specs=[
          pl.BlockSpec((br, K), lambda i: (i, 0)),
          pl.BlockSpec((K, Dout), lambda i: (0, 0)),
          pl.BlockSpec((1, Dout), lambda i: (0, 0)),
      ],
      out_specs=pl.BlockSpec((br, Dout), lambda i: (i, 0)),
      out_shape=jax.ShapeDtypeStruct((R, Dout), out_dtype),
  )(x, w, b.reshape(1, Dout))


def _linear_pack(x, w, b, block_rows=512):
  R, K = x.shape
  Dout = w.shape[1]
  lo, hi = _pack_perms(Dout)
  wcat = jnp.concatenate([w[:, lo], w[:, hi]], axis=1)
  bcat = jnp.concatenate([b[lo], b[hi]]).reshape(1, Dout)

  def body(x_ref, w_ref, b_ref, o_ref):
    z = jnp.dot(x_ref[...], w_ref[...],
                preferred_element_type=jnp.float32) + b_ref[...]
    zl = z[:, :Dout // 2].astype(jnp.bfloat16).astype(jnp.float32)
    zh = z[:, Dout // 2:].astype(jnp.bfloat16).astype(jnp.float32)
    ul = jax.lax.bitcast_convert_type(zl, jnp.uint32)
    uh = jax.lax.bitcast_convert_type(zh, jnp.uint32)
    packed = (uh & jnp.uint32(0xFFFF0000)) | (ul >> 16)
    o_ref[...] = jax.lax.bitcast_convert_type(packed, jnp.int32)

  br = min(block_rows, R)
  return pl.pallas_call(
      body,
      grid=(pl.cdiv(R, br),),
      in_specs=[
          pl.BlockSpec((br, K), lambda i: (i, 0)),
          pl.BlockSpec((K, Dout), lambda i: (0, 0)),
          pl.BlockSpec((1, Dout), lambda i: (0, 0)),
      ],
      out_specs=pl.BlockSpec((br, Dout // 2), lambda i: (i, 0)),
      out_shape=jax.ShapeDtypeStruct((R, Dout // 2), jnp.int32),
  )(x, wcat, bcat)


def _qour_sc(ndi, tdi, qn, qt):
  @functools.partial(
      pl.kernel,
      out_type=jax.ShapeDtypeStruct((_NP, D_OUT), jnp.float32),
      mesh=_vector_mesh,
      scratch_types=[pltpu.VMEM((_CQ, D_OUT), jnp.float32)],
      compiler_params=_sc_params,
  )
  def k(ndi_hbm, tdi_hbm, qn_hbm, qt_hbm, out_hbm, qt_s):
    def body(ndi_v, tdi_v, out_v):
      pltpu.sync_copy(qn_hbm.at[ndi_v.at[0]], out_v)
      pltpu.sync_copy(qt_hbm.at[tdi_v.at[0]], qt_s)

      @pl.loop(0, _CQ)
      def _(r):
        for j in range(D_OUT // 16):
          sl = pl.ds(j * 16, 16)
          out_v[r, sl] += qt_s[r, sl]

    pltpu.emit_pipeline(
        body,
        grid=(_NP // _CQ,),
        in_specs=[
            pl.BlockSpec((1, _CQ), lambda i: (0, i)),
            pl.BlockSpec((1, _CQ), lambda i: (0, i)),
        ],
        out_specs=[pl.BlockSpec((_CQ, D_OUT), lambda i: (i, 0))],
        core_axis_name=("core", "subcore"),
        dimension_semantics=(pltpu.PARALLEL,),
    )(ndi_hbm, tdi_hbm, out_hbm)

  return k(ndi, tdi, qn, qt)


_NW = 32
_C = 64


def _edge_sc(idx1, ni1, ei1, ti1, qour, zn, ze, zt):
  NCT = E // (2 * _C)

  @functools.partial(
      pl.kernel,
      out_type=(jax.ShapeDtypeStruct((H, E), jnp.float32),
                jax.ShapeDtypeStruct((E, D_OUT), jnp.float32)),
      mesh=_vector_mesh,
      scratch_types=[
          [pltpu.VMEM((4 * _C,), jnp.int32) for _ in range(4)],
          [pltpu.VMEM((_C, D_OUT), jnp.float32) for _ in range(2)],
          [pltpu.VMEM((_C, D_OUT), jnp.int32) for _ in range(2)],
          [pltpu.VMEM((_C, D_OUT), jnp.int32) for _ in range(2)],
          [pltpu.VMEM((_C, D_OUT), jnp.int32) for _ in range(2)],
          pltpu.VMEM((4 * _C, D_OUT), jnp.float32),
          pltpu.VMEM((H, 4 * _C), jnp.float32),
          [pltpu.SemaphoreType.DMA for _ in range(2)],
          [pltpu.SemaphoreType.DMA for _ in range(2)],
          pltpu.SemaphoreType.DMA,
      ],
      compiler_params=_sc_params,
  )
  def k(idx_hbm, ni_hbm, ei_hbm, ti_hbm, qour_hbm, zn_hbm, ze_hbm, zt_hbm,
        attn_hbm, v_hbm, ib, qb, znb, zeb, ztb, vt, at, gsem, osem, isem):
    w = jax.lax.axis_index("subcore") * 2 + jax.lax.axis_index("core")
    lo = (NCT * w) // 32
    n = (NCT * (w + 1)) // 32 - lo
    CH = 2 * _C

    def idx_descs(c, ioff):
      return [
          pltpu.make_async_copy(src.at[pl.ds(c * CH, CH)],
                                dst.at[pl.ds(ioff, CH)], isem)
          for src, dst in zip((idx_hbm, ni_hbm, ei_hbm, ti_hbm), ib)
      ]

    def gath(ioff, s, slot):
      def sl(a):
        return a.at[pl.ds(ioff + _C * s, _C)]
      return [
          pltpu.make_async_copy(qour_hbm.at[sl(ib[0])], qb[slot], gsem[slot]),
          pltpu.make_async_copy(zn_hbm.at[sl(ib[1])], znb[slot], gsem[slot]),
          pltpu.make_async_copy(ze_hbm.at[sl(ib[2])], zeb[slot], gsem[slot]),
          pltpu.make_async_copy(zt_hbm.at[sl(ib[3])], ztb[slot], gsem[slot]),
      ]

    def outs(c, p):
      return [
          pltpu.make_async_copy(vt.at[pl.ds(CH * p, CH)],
                                v_hbm.at[pl.ds(c * CH, CH)], osem[p]),
          pltpu.make_async_copy(at.at[:, pl.ds(CH * p, CH)],
                                attn_hbm.at[:, pl.ds(c * CH, CH)], osem[p]),
      ]

    lane = jax.lax.iota(jnp.int32, 16)
    prow = lane >> 3
    pcol = lane & 7

    def compute(slot, colbase):
      q_s, zn_s, ze_s, zt_s = qb[slot], znb[slot], zeb[slot], ztb[slot]

      def unp(x):
        return plsc.unpack(plsc.bitcast(x, jnp.bfloat16),
                           format=plsc.PackFormat.INTERLEAVED)

      @pl.loop(0, _C // 2)
      def _(i):
        acc = jnp.zeros((16,), jnp.float32)
        for e, hbase in ((2 * i, 0), (2 * i + 1, H)):
          for j in range(D_OUT // 32):
            sk = pl.ds(16 * j, 16)
            qa = q_s[e, pl.ds(32 * j, 16)]
            qc = q_s[e, pl.ds(32 * j + 16, 16)]
            na, nc = unp(zn_s[e, sk])
            ea, ec = unp(ze_s[e, sk])
            ta, tc = unp(zt_s[e, sk])
            s0 = jnp.sum(qa * (na + ea + ta))
            s1 = jnp.sum(qc * (nc + ec + tc))
            acc = jnp.where(lane == hbase + 2 * j, s0, acc)
            acc = jnp.where(lane == hbase + 2 * j + 1, s1, acc)
          for j in range(D_OUT // 32):
            sv = pl.ds(D_OUT // 2 + 16 * j, 16)
            na, nc = unp(zn_s[e, sv])
            ea, ec = unp(ze_s[e, sv])
            ta, tc = unp(zt_s[e, sv])
            vt[colbase + e, pl.ds(32 * j, 16)] = na + ea + ta
            vt[colbase + e, pl.ds(32 * j + 16, 16)] = nc + ec + tc
        acc = jnp.where(acc >= 0., acc, 0.2 * acc)
        plsc.store_scatter(at, [pcol, colbase + 2 * i + prow], acc)

    for d in idx_descs(lo, 0):
      d.start()
    for d in idx_descs(lo, 0):
      d.wait()
    for d in gath(0, 0, 0):
      d.start()

    @pl.loop(0, n)
    def _(lc):
      c = lo + lc
      par = lc & 1
      ioff = CH * par

      @pl.when(lc + 1 < n)
      def _():
        for d in idx_descs(c + 1, CH * (1 - par)):
          d.start()

      for s in range(2):
        for d in gath(ioff, s, s):
          d.wait()
        if s == 0:
          for d in gath(ioff, 1, 1):
            d.start()

          @pl.when(lc >= 2)
          def _():
            for p in range(2):
              @pl.when(par == p)
              def _(p=p):
                for d in outs(c - 2, p):
                  d.wait()
        else:
          @pl.when(lc + 1 < n)
          def _():
            for d in idx_descs(c + 1, CH * (1 - par)):
              d.wait()
            for d in gath(CH * (1 - par), 0, 0):
              d.start()
        compute(s, ioff + _C * s)

      for p in range(2):
        @pl.when(par == p)
        def _(p=p):
          for d in outs(c, p):
            d.start()

    for k2 in (2, 1):
      @pl.when(n >= k2)
      def _(k2=k2):
        for p in range(2):
          @pl.when(((n - k2) & 1) == p)
          def _(k2=k2, p=p):
            for d in outs(lo + n - k2, p):
              d.wait()

  return k(idx1, ni1, ei1, ti1, qour, zn, ze, zt)


def kernel(idx, nodeData, node_inverse, node_dst_inverse, efeat_unique,
           efeat_inverse, time_unique, time_inverse, time_dst_unique,
           time_dst_inverse, W_q_node, b_q_node, W_q_time, b_q_time,
           W_kv_node, b_kv_node, W_kv_edge, b_kv_edge, W_kv_time, b_kv_time):
  qn = _linear(nodeData, W_q_node, b_q_node)
  qt = _linear(time_dst_unique, W_q_time, b_q_time)
  znode = _linear_pack(nodeData, W_kv_node, b_kv_node)
  zedge = _linear_pack(efeat_unique, W_kv_edge, b_kv_edge)
  ztime = _linear_pack(time_unique, W_kv_time, b_kv_time)

  pad = _NP - N
  ndi = jnp.pad(node_dst_inverse, (0, pad)).reshape(1, _NP)
  tdi = jnp.pad(time_dst_inverse, (0, pad)).reshape(1, _NP)
  qour = _qour_sc(ndi, tdi, qn, qt)

  attn_t, v_t = _edge_sc(
      idx, node_inverse, efeat_inverse, time_inverse,
      qour, znode, zedge, ztime)
  return (attn_t.T, v_t.reshape(E, H, D_OUT // H))


# speedup vs baseline: 1.0020x; 1.0020x over previous
import dataclasses
import functools

import jax
import jax.numpy as jnp
import numpy as np
from jax.experimental import pallas as pl
from jax.experimental.pallas import tpu as pltpu
from jax.experimental.pallas import tpu_sc as plsc

N = 10000
E = 320000
D_OUT = 128
H = 8

_NP = 10240
_CQ = 128

def _vmesh():
  return plsc.VectorSubcoreMesh(
      core_axis_name="core", subcore_axis_name="subcore")

_sc_params = pltpu.CompilerParams()
if "needs_layout_passes" in pltpu.CompilerParams.__dataclass_fields__:
  _sc_params = dataclasses.replace(_sc_params, needs_layout_passes=False)


def _pack_perms(width):
  q = np.arange(width // 2)
  lo = 32 * (q // 16) + q % 16
  return np.asarray(lo, np.int32), np.asarray(lo + 16, np.int32)


def _linear(x, w, b, out_dtype=jnp.float32, block_rows=512):
  R, K = x.shape
  Dout = w.shape[1]

  def body(x_ref, w_ref, b_ref, o_ref):
    o_ref[...] = (jnp.dot(x_ref[...], w_ref[...],
                          preferred_element_type=jnp.float32)
                  + b_ref[...]).astype(out_dtype)

  br = min(block_rows, R)
  return pl.pallas_call(
      body,
      grid=(pl.cdiv(R, br),),
      in_specs=[
          pl.BlockSpec((br, K), lambda i: (i, 0)),
          pl.BlockSpec((K, Dout), lambda i: (0, 0)),
          pl.BlockSpec((1, Dout), lambda i: (0, 0)),
      ],
      out_specs=pl.BlockSpec((br, Dout), lambda i: (i, 0)),
      out_shape=jax.ShapeDtypeStruct((R, Dout), out_dtype),
  )(x, w, b.reshape(1, Dout))


def _linear_pack(x, w, b, block_rows=512):
  R, K = x.shape
  Dout = w.shape[1]
  lo, hi = _pack_perms(Dout)
  wcat = jnp.concatenate([w[:, lo], w[:, hi]], axis=1)
  bcat = jnp.concatenate([b[lo], b[hi]]).reshape(1, Dout)

  def body(x_ref, w_ref, b_ref, o_ref):
    z = jnp.dot(x_ref[...], w_ref[...],
                preferred_element_type=jnp.float32) + b_ref[...]
    zl = z[:, :Dout // 2].astype(jnp.bfloat16).astype(jnp.float32)
    zh = z[:, Dout // 2:].astype(jnp.bfloat16).astype(jnp.float32)
    ul = jax.lax.bitcast_convert_type(zl, jnp.uint32)
    uh = jax.lax.bitcast_convert_type(zh, jnp.uint32)
    packed = (uh & jnp.uint32(0xFFFF0000)) | (ul >> 16)
    o_ref[...] = jax.lax.bitcast_convert_type(packed, jnp.int32)

  br = min(block_rows, R)
  return pl.pallas_call(
      body,
      grid=(pl.cdiv(R, br),),
      in_specs=[
          pl.BlockSpec((br, K), lambda i: (i, 0)),
          pl.BlockSpec((K, Dout), lambda i: (0, 0)),
          pl.BlockSpec((1, Dout), lambda i: (0, 0)),
      ],
      out_specs=pl.BlockSpec((br, Dout // 2), lambda i: (i, 0)),
      out_shape=jax.ShapeDtypeStruct((R, Dout // 2), jnp.int32),
  )(x, wcat, bcat)


def _qour_sc(ndi, tdi, qn, qt):
  @functools.partial(
      pl.kernel,
      out_type=jax.ShapeDtypeStruct((_NP, D_OUT), jnp.float32),
      mesh=_vmesh(),
      scratch_types=[pltpu.VMEM((_CQ, D_OUT), jnp.float32)],
      compiler_params=_sc_params,
  )
  def k(ndi_hbm, tdi_hbm, qn_hbm, qt_hbm, out_hbm, qt_s):
    def body(ndi_v, tdi_v, out_v):
      pltpu.sync_copy(qn_hbm.at[ndi_v.at[0]], out_v)
      pltpu.sync_copy(qt_hbm.at[tdi_v.at[0]], qt_s)

      @pl.loop(0, _CQ)
      def _(r):
        for j in range(D_OUT // 16):
          sl = pl.ds(j * 16, 16)
          out_v[r, sl] += qt_s[r, sl]

    pltpu.emit_pipeline(
        body,
        grid=(_NP // _CQ,),
        in_specs=[
            pl.BlockSpec((1, _CQ), lambda i: (0, i)),
            pl.BlockSpec((1, _CQ), lambda i: (0, i)),
        ],
        out_specs=[pl.BlockSpec((_CQ, D_OUT), lambda i: (i, 0))],
        core_axis_name=("core", "subcore"),
        dimension_semantics=(pltpu.PARALLEL,),
    )(ndi_hbm, tdi_hbm, out_hbm)

  return k(ndi, tdi, qn, qt)


_NW = 32
_C = 64


def _edge_sc(idx1, ni1, ei1, ti1, qour, zn, ze, zt):
  NCT = E // (2 * _C)

  @functools.partial(
      pl.kernel,
      out_type=(jax.ShapeDtypeStruct((H, E), jnp.float32),
                jax.ShapeDtypeStruct((E, D_OUT), jnp.float32)),
      mesh=_vmesh(),
      scratch_types=[
          [pltpu.VMEM((4 * _C,), jnp.int32) for _ in range(4)],
          [pltpu.VMEM((_C, D_OUT), jnp.float32) for _ in range(2)],
          [pltpu.VMEM((_C, D_OUT), jnp.int32) for _ in range(2)],
          [pltpu.VMEM((_C, D_OUT), jnp.int32) for _ in range(2)],
          [pltpu.VMEM((_C, D_OUT), jnp.int32) for _ in range(2)],
          pltpu.VMEM((4 * _C, D_OUT), jnp.float32),
          pltpu.VMEM((H, 4 * _C), jnp.float32),
          [pltpu.SemaphoreType.DMA for _ in range(2)],
          [pltpu.SemaphoreType.DMA for _ in range(2)],
          pltpu.SemaphoreType.DMA,
      ],
      compiler_params=_sc_params,
  )
  def k(idx_hbm, ni_hbm, ei_hbm, ti_hbm, qour_hbm, zn_hbm, ze_hbm, zt_hbm,
        attn_hbm, v_hbm, ib, qb, znb, zeb, ztb, vt, at, gsem, osem, isem):
    w = jax.lax.axis_index("subcore") * 2 + jax.lax.axis_index("core")
    lo = (NCT * w) // 32
    n = (NCT * (w + 1)) // 32 - lo
    CH = 2 * _C

    def idx_descs(c, ioff):
      return [
          pltpu.make_async_copy(src.at[pl.ds(c * CH, CH)],
                                dst.at[pl.ds(ioff, CH)], isem)
          for src, dst in zip((idx_hbm, ni_hbm, ei_hbm, ti_hbm), ib)
      ]

    def gath(ioff, s, slot):
      def sl(a):
        return a.at[pl.ds(ioff + _C * s, _C)]
      return [
          pltpu.make_async_copy(qour_hbm.at[sl(ib[0])], qb[slot], gsem[slot]),
          pltpu.make_async_copy(zn_hbm.at[sl(ib[1])], znb[slot], gsem[slot]),
          pltpu.make_async_copy(ze_hbm.at[sl(ib[2])], zeb[slot], gsem[slot]),
          pltpu.make_async_copy(zt_hbm.at[sl(ib[3])], ztb[slot], gsem[slot]),
      ]

    def outs(c, p):
      return [
          pltpu.make_async_copy(vt.at[pl.ds(CH * p, CH)],
                                v_hbm.at[pl.ds(c * CH, CH)], osem[p]),
          pltpu.make_async_copy(at.at[:, pl.ds(CH * p, CH)],
                                attn_hbm.at[:, pl.ds(c * CH, CH)], osem[p]),
      ]

    lane = jax.lax.iota(jnp.int32, 16)
    prow = lane >> 3
    pcol = lane & 7

    def compute(slot, colbase):
      q_s, zn_s, ze_s, zt_s = qb[slot], znb[slot], zeb[slot], ztb[slot]

      def unp(x):
        return plsc.unpack(plsc.bitcast(x, jnp.bfloat16),
                           format=plsc.PackFormat.INTERLEAVED)

      @pl.loop(0, _C // 2)
      def _(i):
        acc = jnp.zeros((16,), jnp.float32)
        for e, hbase in ((2 * i, 0), (2 * i + 1, H)):
          for j in range(D_OUT // 32):
            sk = pl.ds(16 * j, 16)
            qa = q_s[e, pl.ds(32 * j, 16)]
            qc = q_s[e, pl.ds(32 * j + 16, 16)]
            na, nc = unp(zn_s[e, sk])
            ea, ec = unp(ze_s[e, sk])
            ta, tc = unp(zt_s[e, sk])
            s0 = jnp.sum(qa * (na + ea + ta))
            s1 = jnp.sum(qc * (nc + ec + tc))
            acc = jnp.where(lane == hbase + 2 * j, s0, acc)
            acc = jnp.where(lane == hbase + 2 * j + 1, s1, acc)
          for j in range(D_OUT // 32):
            sv = pl.ds(D_OUT // 2 + 16 * j, 16)
            na, nc = unp(zn_s[e, sv])
            ea, ec = unp(ze_s[e, sv])
            ta, tc = unp(zt_s[e, sv])
            vt[colbase + e, pl.ds(32 * j, 16)] = na + ea + ta
            vt[colbase + e, pl.ds(32 * j + 16, 16)] = nc + ec + tc
        acc = jnp.where(acc >= 0., acc, 0.2 * acc)
        plsc.store_scatter(at, [pcol, colbase + 2 * i + prow], acc)

    for d in idx_descs(lo, 0):
      d.start()
    for d in idx_descs(lo, 0):
      d.wait()
    for d in gath(0, 0, 0):
      d.start()

    @pl.loop(0, n)
    def _(lc):
      c = lo + lc
      par = lc & 1
      ioff = CH * par

      @pl.when(lc + 1 < n)
      def _():
        for d in idx_descs(c + 1, CH * (1 - par)):
          d.start()

      for s in range(2):
        for d in gath(ioff, s, s):
          d.wait()
        if s == 0:
          for d in gath(ioff, 1, 1):
            d.start()

          @pl.when(lc >= 2)
          def _():
            for p in range(2):
              @pl.when(par == p)
              def _(p=p):
                for d in outs(c - 2, p):
                  d.wait()
        else:
          @pl.when(lc + 1 < n)
          def _():
            for d in idx_descs(c + 1, CH * (1 - par)):
              d.wait()
            for d in gath(CH * (1 - par), 0, 0):
              d.start()
        compute(s, ioff + _C * s)

      for p in range(2):
        @pl.when(par == p)
        def _(p=p):
          for d in outs(c, p):
            d.start()

    for k2 in (2, 1):
      @pl.when(n >= k2)
      def _(k2=k2):
        for p in range(2):
          @pl.when(((n - k2) & 1) == p)
          def _(k2=k2, p=p):
            for d in outs(lo + n - k2, p):
              d.wait()

  return k(idx1, ni1, ei1, ti1, qour, zn, ze, zt)


def kernel(idx, nodeData, node_inverse, node_dst_inverse, efeat_unique,
           efeat_inverse, time_unique, time_inverse, time_dst_unique,
           time_dst_inverse, W_q_node, b_q_node, W_q_time, b_q_time,
           W_kv_node, b_kv_node, W_kv_edge, b_kv_edge, W_kv_time, b_kv_time):
  qn = _linear(nodeData, W_q_node, b_q_node)
  qt = _linear(time_dst_unique, W_q_time, b_q_time)
  znode = _linear_pack(nodeData, W_kv_node, b_kv_node)
  zedge = _linear_pack(efeat_unique, W_kv_edge, b_kv_edge)
  ztime = _linear_pack(time_unique, W_kv_time, b_kv_time)

  pad = _NP - N
  ndi = jnp.pad(node_dst_inverse, (0, pad)).reshape(1, _NP)
  tdi = jnp.pad(time_dst_inverse, (0, pad)).reshape(1, _NP)
  qour = _qour_sc(ndi, tdi, qn, qt)

  attn_t, v_t = _edge_sc(
      idx, node_inverse, efeat_inverse, time_inverse,
      qour, znode, zedge, ztime)
  return (attn_t.T, v_t.reshape(E, H, D_OUT // H))
